# baseline (speedup 1.0000x reference)
import dataclasses
import functools

import jax
import jax.numpy as jnp
import numpy as np
from jax import lax
from jax.experimental import pallas as pl
from jax.experimental.pallas import tpu as pltpu
from jax.experimental.pallas import tpu_sc as plsc

N = 4000000
RES = 100000
ROWS = 784
HIST_PAD = ROWS * 128
NW = 32

W_IMP = float(np.float32(1.0) / np.float32(1e8))
SCALE = float(np.float32(RES) / np.float32(1.0))

_TOTAL = 100000000 + 100000000
_AVG_PTS = _TOTAL / 1000
_UW = 1.0 / min(2.0 * _TOTAL, 5000.0 * _AVG_PTS)
CUW = float(np.float32((1.0 - _UW) ** float(N)))

CNT_HI = 125008
CNT_LO = 124992
CH = 1920
N_FULL = 64
REM = 2112
STRIPE = HIST_PAD // 16


def _sc_hist_body(stat_hbm, dyn_hbm, score_hbm, out_hbm,
                  hist_v, stat_v, dyn_v, score_v, shared_v, idx_v,
                  sem0, sem1, sem2, sem3):
    core = lax.axis_index("c")
    sid = lax.axis_index("s")
    wid = sid * 2 + core
    is_hi = wid < 16
    base = jnp.where(is_hi, wid * CNT_HI,
                     16 * CNT_HI + (wid - 16) * CNT_LO)

    zero = jnp.zeros((16,), jnp.float32)
    lane = lax.broadcasted_iota(jnp.int32, (16,), 0)

    @pl.loop(0, ROWS)
    def _(r):
        for u in range(8):
            hist_v[r, pl.ds(u * 16, 16)] = zero

    @pl.loop(0, ROWS // 16)
    def _(i):
        idx_v[pl.ds(i * 16, 16)] = lane + i * 16

    sbase = pl.multiple_of(
        jnp.where(sid < 14, sid * 48, 672 + (sid - 14) * 56), 8)

    pltpu.sync_copy(hist_v.at[pl.ds(0, 48)], shared_v.at[pl.ds(sbase, 48)])

    @pl.when(sid >= 14)
    def _():
        pltpu.sync_copy(hist_v.at[pl.ds(0, 8)],
                        shared_v.at[pl.ds(sbase + 48, 8)])

    plsc.subcore_barrier()

    def scatter_vreg(a, b, s):
        val = a - b
        idx = lax.convert_element_type(s * np.float32(SCALE), jnp.int32)
        idx = jnp.minimum(jnp.maximum(idx, 0), RES - 1)
        plsc.addupdate_scatter(
            hist_v,
            [lax.shift_right_logical(idx, 7), jnp.bitwise_and(idx, 127)],
            val)

    def compute(slot, nvreg, unroll):
        @plsc.parallel_loop(0, nvreg * 16, 16, unroll=unroll)
        def _(o):
            scatter_vreg(stat_v[pl.ds(slot * CH + o, 16)],
                         dyn_v[pl.ds(slot * CH + o, 16)],
                         score_v[pl.ds(slot * CH + o, 16)])

    def copies(slot, c, sem):
        off = base + c * CH
        return [
            pltpu.make_async_copy(stat_hbm.at[pl.ds(off, CH)],
                                  stat_v.at[pl.ds(slot * CH, CH)], sem),
            pltpu.make_async_copy(dyn_hbm.at[pl.ds(off, CH)],
                                  dyn_v.at[pl.ds(slot * CH, CH)], sem),
            pltpu.make_async_copy(score_hbm.at[pl.ds(off, CH)],
                                  score_v.at[pl.ds(slot * CH, CH)], sem),
        ]

    def start(slot, c, sem):
        for cp in copies(slot, c, sem):
            cp.start()

    def wait(slot, c, sem):
        for cp in copies(slot, c, sem):
            cp.wait()

    sems = [sem0, sem1, sem2, sem3]
    for b in range(4):
        start(b, b, sems[b])

    @pl.loop(0, N_FULL // 4)
    def _(i):
        for b in range(4):
            wait(b, 4 * i + b, sems[b])
            compute(b, CH // 16, 16)

            @pl.when(i < N_FULL // 4 - 1)
            def _():
                start(b, 4 * i + b + 4, sems[b])

    off = base + N_FULL * CH
    pltpu.sync_copy(stat_hbm.at[pl.ds(off, REM)], stat_v.at[pl.ds(0, REM)])
    pltpu.sync_copy(dyn_hbm.at[pl.ds(off, REM)], dyn_v.at[pl.ds(0, REM)])
    pltpu.sync_copy(score_hbm.at[pl.ds(off, REM)], score_v.at[pl.ds(0, REM)])
    compute(0, REM // 16, 4)

    @pl.when(is_hi)
    def _():
        off2 = base + N_FULL * CH + REM
        pltpu.sync_copy(stat_hbm.at[pl.ds(off2, 16)], stat_v.at[pl.ds(0, 16)])
        pltpu.sync_copy(dyn_hbm.at[pl.ds(off2, 16)], dyn_v.at[pl.ds(0, 16)])
        pltpu.sync_copy(score_hbm.at[pl.ds(off2, 16)], score_v.at[pl.ds(0, 16)])
        scatter_vreg(stat_v[pl.ds(0, 16)], dyn_v[pl.ds(0, 16)],
                     score_v[pl.ds(0, 16)])

    pltpu.sync_copy(hist_v, shared_v.at[idx_v], add=True)
    plsc.subcore_barrier()
    pltpu.sync_copy(shared_v.at[pl.ds(sbase, 48)],
                    out_hbm.at[core, pl.ds(sbase, 48)])

    @pl.when(sid >= 14)
    def _():
        pltpu.sync_copy(shared_v.at[pl.ds(sbase + 48, 8)],
                        out_hbm.at[core, pl.ds(sbase + 48, 8)])


@jax.jit
def _sc_hist(stat, dyn, score):
    mesh = plsc.VectorSubcoreMesh(core_axis_name="c", subcore_axis_name="s")
    cp = pltpu.CompilerParams()
    if "needs_layout_passes" in pltpu.CompilerParams.__dataclass_fields__:
        cp = dataclasses.replace(cp, needs_layout_passes=False)
    f = pl.kernel(
        _sc_hist_body,
        out_type=jax.ShapeDtypeStruct((2, ROWS, 128), jnp.float32),
        mesh=mesh,
        scratch_types=[
            pltpu.VMEM((ROWS, 128), jnp.float32),
            pltpu.VMEM((4 * CH,), jnp.float32),
            pltpu.VMEM((4 * CH,), jnp.float32),
            pltpu.VMEM((4 * CH,), jnp.float32),
            pltpu.VMEM_SHARED((ROWS, 128), jnp.float32),
            pltpu.VMEM((ROWS,), jnp.int32),
            pltpu.SemaphoreType.DMA,
            pltpu.SemaphoreType.DMA,
            pltpu.SemaphoreType.DMA,
            pltpu.SemaphoreType.DMA,
        ],
        compiler_params=cp,
    )
    return f(stat, dyn, score)


def _tc_post_body(ph_ref, mai_ref, out_ref):
    h = jnp.sum(ph_ref[...], axis=0)
    mai = mai_ref[...] * np.float32(CUW) + (np.float32(1.0 - CUW) * np.float32(W_IMP)) * h

    ii = lax.broadcasted_iota(jnp.int32, (128, 128), 0)
    jj = lax.broadcasted_iota(jnp.int32, (128, 128), 1)
    upper = (ii <= jj).astype(jnp.float32)
    w = lax.dot_general(mai, upper, (((1,), (0,)), ((), ())),
                        preferred_element_type=jnp.float32,
                        precision=lax.Precision.HIGHEST)

    rr = lax.broadcasted_iota(jnp.int32, (ROWS, ROWS), 0)
    cc = lax.broadcasted_iota(jnp.int32, (ROWS, ROWS), 1)
    lstrict = (cc < rr).astype(jnp.float32)
    s_b = jnp.broadcast_to(w[:, 127:128], (ROWS, 128))
    p = lax.dot_general(lstrict, s_b, (((1,), (0,)), ((), ())),
                        preferred_element_type=jnp.float32,
                        precision=lax.Precision.HIGHEST)
    c = w + p

    r2 = lax.broadcasted_iota(jnp.int32, (ROWS, 128), 0)
    l2 = lax.broadcasted_iota(jnp.int32, (ROWS, 128), 1)
    k = r2 * 128 + l2
    valid = k < RES
    cv = jnp.where(valid, c, jnp.float32(jnp.inf))
    best = jnp.minimum(jnp.min(cv), jnp.float32(0.0))
    eq = cv == best
    cnt = jnp.sum(eq.astype(jnp.float32)) + (best == 0.0).astype(jnp.float32)
    idxsum = jnp.sum(jnp.where(eq, (k + 1).astype(jnp.float32), jnp.float32(0.0)))
    avg = idxsum / cnt
    out_ref[...] = jnp.broadcast_to(avg * np.float32(1.0) / np.float32(RES), (1, 1))


@jax.jit
def _tc_post(part, mai_pad):
    return pl.pallas_call(
        _tc_post_body,
        out_shape=jax.ShapeDtypeStruct((1, 1), jnp.float32),
    )(part.reshape(-1, ROWS, 128), mai_pad)


def kernel(epes_stat_flow, epes_dyn_flow, moving_mask, dynamicness_scores,
           moving_average_importance, training=True):
    part = _sc_hist(epes_stat_flow, epes_dyn_flow, dynamicness_scores)
    mai_pad = jnp.pad(moving_average_importance, (0, HIST_PAD - RES)).reshape(ROWS, 128)
    out = _tc_post(part, mai_pad)
    return out[0, 0]


# speedup vs baseline: 1.0949x; 1.0949x over previous
import dataclasses
import functools

import jax
import jax.numpy as jnp
import numpy as np
from jax import lax
from jax.experimental import pallas as pl
from jax.experimental.pallas import tpu as pltpu
from jax.experimental.pallas import tpu_sc as plsc

N = 4000000
RES = 100000
ROWS = 784
HIST_PAD = ROWS * 128
NW = 32

W_IMP = float(np.float32(1.0) / np.float32(1e8))
SCALE = float(np.float32(RES) / np.float32(1.0))

_TOTAL = 100000000 + 100000000
_AVG_PTS = _TOTAL / 1000
_UW = 1.0 / min(2.0 * _TOTAL, 5000.0 * _AVG_PTS)
CUW = float(np.float32((1.0 - _UW) ** float(N)))

CNT_HI = 125008
CNT_LO = 124992
CH = 2560
N_FULL = 48
REM = 2112
STRIPE = HIST_PAD // 16


def _sc_hist_body(stat_hbm, dyn_hbm, score_hbm, out_hbm,
                  hist_v, stat_v, dyn_v, score_v, shared_v, idx_v,
                  sem0, sem1, sem2):
    core = lax.axis_index("c")
    sid = lax.axis_index("s")
    wid = sid * 2 + core
    is_hi = wid < 16
    base = jnp.where(is_hi, wid * CNT_HI,
                     16 * CNT_HI + (wid - 16) * CNT_LO)

    def copies(slot, c, sem):
        off = base + c * CH
        return [
            pltpu.make_async_copy(stat_hbm.at[pl.ds(off, CH)],
                                  stat_v.at[pl.ds(slot * CH, CH)], sem),
            pltpu.make_async_copy(dyn_hbm.at[pl.ds(off, CH)],
                                  dyn_v.at[pl.ds(slot * CH, CH)], sem),
            pltpu.make_async_copy(score_hbm.at[pl.ds(off, CH)],
                                  score_v.at[pl.ds(slot * CH, CH)], sem),
        ]

    def start(slot, c, sem):
        for cp in copies(slot, c, sem):
            cp.start()

    def wait(slot, c, sem):
        for cp in copies(slot, c, sem):
            cp.wait()

    sems = [sem0, sem1, sem2]
    for b in range(3):
        start(b, b, sems[b])

    zero = jnp.zeros((16,), jnp.float32)
    lane = lax.broadcasted_iota(jnp.int32, (16,), 0)

    @pl.loop(0, ROWS)
    def _(r):
        for u in range(8):
            hist_v[r, pl.ds(u * 16, 16)] = zero

    @pl.loop(0, ROWS // 16)
    def _(i):
        idx_v[pl.ds(i * 16, 16)] = lane + i * 16

    sbase = pl.multiple_of(
        jnp.where(sid < 14, sid * 48, 672 + (sid - 14) * 56), 8)

    pltpu.sync_copy(hist_v.at[pl.ds(0, 48)], shared_v.at[pl.ds(sbase, 48)])

    @pl.when(sid >= 14)
    def _():
        pltpu.sync_copy(hist_v.at[pl.ds(0, 8)],
                        shared_v.at[pl.ds(sbase + 48, 8)])

    plsc.subcore_barrier()

    def scatter_vreg(a, b, s):
        val = a - b
        idx = lax.convert_element_type(s * np.float32(SCALE), jnp.int32)
        idx = jnp.minimum(jnp.maximum(idx, 0), RES - 1)
        plsc.addupdate_scatter(
            hist_v,
            [lax.shift_right_logical(idx, 7), jnp.bitwise_and(idx, 127)],
            val)

    def compute(slot, nvreg, unroll):
        @plsc.parallel_loop(0, nvreg * 16, 16, unroll=unroll)
        def _(o):
            scatter_vreg(stat_v[pl.ds(slot * CH + o, 16)],
                         dyn_v[pl.ds(slot * CH + o, 16)],
                         score_v[pl.ds(slot * CH + o, 16)])

    @pl.loop(0, N_FULL // 3)
    def _(i):
        for b in range(3):
            wait(b, 3 * i + b, sems[b])
            compute(b, CH // 16, 16)

            @pl.when(i < N_FULL // 3 - 1)
            def _():
                start(b, 3 * i + b + 3, sems[b])

    off = base + N_FULL * CH
    pltpu.sync_copy(stat_hbm.at[pl.ds(off, REM)], stat_v.at[pl.ds(0, REM)])
    pltpu.sync_copy(dyn_hbm.at[pl.ds(off, REM)], dyn_v.at[pl.ds(0, REM)])
    pltpu.sync_copy(score_hbm.at[pl.ds(off, REM)], score_v.at[pl.ds(0, REM)])
    compute(0, REM // 16, 4)

    @pl.when(is_hi)
    def _():
        off2 = base + N_FULL * CH + REM
        pltpu.sync_copy(stat_hbm.at[pl.ds(off2, 16)], stat_v.at[pl.ds(0, 16)])
        pltpu.sync_copy(dyn_hbm.at[pl.ds(off2, 16)], dyn_v.at[pl.ds(0, 16)])
        pltpu.sync_copy(score_hbm.at[pl.ds(off2, 16)], score_v.at[pl.ds(0, 16)])
        scatter_vreg(stat_v[pl.ds(0, 16)], dyn_v[pl.ds(0, 16)],
                     score_v[pl.ds(0, 16)])

    pltpu.sync_copy(hist_v, shared_v.at[idx_v], add=True)
    plsc.subcore_barrier()
    pltpu.sync_copy(shared_v.at[pl.ds(sbase, 48)],
                    out_hbm.at[core, pl.ds(sbase, 48)])

    @pl.when(sid >= 14)
    def _():
        pltpu.sync_copy(shared_v.at[pl.ds(sbase + 48, 8)],
                        out_hbm.at[core, pl.ds(sbase + 48, 8)])


@jax.jit
def _sc_hist(stat, dyn, score):
    mesh = plsc.VectorSubcoreMesh(core_axis_name="c", subcore_axis_name="s")
    cp = pltpu.CompilerParams()
    if "needs_layout_passes" in pltpu.CompilerParams.__dataclass_fields__:
        cp = dataclasses.replace(cp, needs_layout_passes=False)
    f = pl.kernel(
        _sc_hist_body,
        out_type=jax.ShapeDtypeStruct((2, ROWS, 128), jnp.float32),
        mesh=mesh,
        scratch_types=[
            pltpu.VMEM((ROWS, 128), jnp.float32),
            pltpu.VMEM((3 * CH,), jnp.float32),
            pltpu.VMEM((3 * CH,), jnp.float32),
            pltpu.VMEM((3 * CH,), jnp.float32),
            pltpu.VMEM_SHARED((ROWS, 128), jnp.float32),
            pltpu.VMEM((ROWS,), jnp.int32),
            pltpu.SemaphoreType.DMA,
            pltpu.SemaphoreType.DMA,
            pltpu.SemaphoreType.DMA,
        ],
        compiler_params=cp,
    )
    return f(stat, dyn, score)


def _tc_post_body(ph_ref, mai_ref, out_ref):
    h = jnp.sum(ph_ref[...], axis=0)
    mai = mai_ref[...] * np.float32(CUW) + (np.float32(1.0 - CUW) * np.float32(W_IMP)) * h

    ii = lax.broadcasted_iota(jnp.int32, (128, 128), 0)
    jj = lax.broadcasted_iota(jnp.int32, (128, 128), 1)
    upper = (ii <= jj).astype(jnp.float32)
    w = lax.dot_general(mai, upper, (((1,), (0,)), ((), ())),
                        preferred_element_type=jnp.float32,
                        precision=lax.Precision.HIGHEST)

    rr = lax.broadcasted_iota(jnp.int32, (ROWS, ROWS), 0)
    cc = lax.broadcasted_iota(jnp.int32, (ROWS, ROWS), 1)
    lstrict = (cc < rr).astype(jnp.float32)
    s_b = jnp.broadcast_to(w[:, 127:128], (ROWS, 128))
    p = lax.dot_general(lstrict, s_b, (((1,), (0,)), ((), ())),
                        preferred_element_type=jnp.float32,
                        precision=lax.Precision.HIGHEST)
    c = w + p

    r2 = lax.broadcasted_iota(jnp.int32, (ROWS, 128), 0)
    l2 = lax.broadcasted_iota(jnp.int32, (ROWS, 128), 1)
    k = r2 * 128 + l2
    valid = k < RES
    cv = jnp.where(valid, c, jnp.float32(jnp.inf))
    best = jnp.minimum(jnp.min(cv), jnp.float32(0.0))
    eq = cv == best
    cnt = jnp.sum(eq.astype(jnp.float32)) + (best == 0.0).astype(jnp.float32)
    idxsum = jnp.sum(jnp.where(eq, (k + 1).astype(jnp.float32), jnp.float32(0.0)))
    avg = idxsum / cnt
    out_ref[...] = jnp.broadcast_to(avg * np.float32(1.0) / np.float32(RES), (1, 1))


@jax.jit
def _tc_post(part, mai_pad):
    return pl.pallas_call(
        _tc_post_body,
        out_shape=jax.ShapeDtypeStruct((1, 1), jnp.float32),
    )(part.reshape(-1, ROWS, 128), mai_pad)


def kernel(epes_stat_flow, epes_dyn_flow, moving_mask, dynamicness_scores,
           moving_average_importance, training=True):
    part = _sc_hist(epes_stat_flow, epes_dyn_flow, dynamicness_scores)
    mai_pad = jnp.pad(moving_average_importance, (0, HIST_PAD - RES)).reshape(ROWS, 128)
    out = _tc_post(part, mai_pad)
    return out[0, 0]


# speedup vs baseline: 1.0957x; 1.0007x over previous
import dataclasses
import functools

import jax
import jax.numpy as jnp
import numpy as np
from jax import lax
from jax.experimental import pallas as pl
from jax.experimental.pallas import tpu as pltpu
from jax.experimental.pallas import tpu_sc as plsc

N = 4000000
RES = 100000
ROWS = 784
HIST_PAD = ROWS * 128
NW = 32

W_IMP = float(np.float32(1.0) / np.float32(1e8))
SCALE = float(np.float32(RES) / np.float32(1.0))

_TOTAL = 100000000 + 100000000
_AVG_PTS = _TOTAL / 1000
_UW = 1.0 / min(2.0 * _TOTAL, 5000.0 * _AVG_PTS)
CUW = float(np.float32((1.0 - _UW) ** float(N)))

CNT_HI = 125008
CNT_LO = 124992
CH = 2560
N_FULL = 48
REM = 2112


def _sc_hist_body(stat_hbm, dyn_hbm, score_hbm, out_hbm,
                  hist_v, stat_v, dyn_v, score_v, shared_v, idx_v,
                  sem0, sem1, sem2):
    core = lax.axis_index("c")
    sid = lax.axis_index("s")
    wid = sid * 2 + core
    is_hi = wid < 16
    base = jnp.where(is_hi, wid * CNT_HI,
                     16 * CNT_HI + (wid - 16) * CNT_LO)

    def copies(slot, c, sem):
        off = base + c * CH
        return [
            pltpu.make_async_copy(stat_hbm.at[pl.ds(off, CH)],
                                  stat_v.at[pl.ds(slot * CH, CH)], sem),
            pltpu.make_async_copy(dyn_hbm.at[pl.ds(off, CH)],
                                  dyn_v.at[pl.ds(slot * CH, CH)], sem),
            pltpu.make_async_copy(score_hbm.at[pl.ds(off, CH)],
                                  score_v.at[pl.ds(slot * CH, CH)], sem),
        ]

    def start(slot, c, sem):
        for cp in copies(slot, c, sem):
            cp.start()

    def wait(slot, c, sem):
        for cp in copies(slot, c, sem):
            cp.wait()

    sems = [sem0, sem1, sem2]
    for b in range(3):
        start(b, b, sems[b])

    zero = jnp.zeros((16,), jnp.float32)
    lane = lax.broadcasted_iota(jnp.int32, (16,), 0)

    @pl.loop(0, ROWS)
    def _(r):
        for u in range(8):
            hist_v[r, pl.ds(u * 16, 16)] = zero

    @pl.loop(0, ROWS // 16)
    def _(i):
        idx_v[pl.ds(i * 16, 16)] = lane + i * 16

    sbase = pl.multiple_of(
        jnp.where(sid < 14, sid * 48, 672 + (sid - 14) * 56), 8)

    pltpu.sync_copy(hist_v.at[pl.ds(0, 48)], shared_v.at[pl.ds(sbase, 48)])

    @pl.when(sid >= 14)
    def _():
        pltpu.sync_copy(hist_v.at[pl.ds(0, 8)],
                        shared_v.at[pl.ds(sbase + 48, 8)])

    plsc.subcore_barrier()

    def scatter_vreg(a, b, s):
        val = a - b
        idx = lax.convert_element_type(s * np.float32(SCALE), jnp.int32)
        idx = jnp.minimum(jnp.maximum(idx, 0), RES - 1)
        plsc.addupdate_scatter(
            hist_v,
            [lax.shift_right_logical(idx, 7), jnp.bitwise_and(idx, 127)],
            val)

    def compute(slot, nvreg, unroll):
        @plsc.parallel_loop(0, nvreg * 16, 16, unroll=unroll)
        def _(o):
            scatter_vreg(stat_v[pl.ds(slot * CH + o, 16)],
                         dyn_v[pl.ds(slot * CH + o, 16)],
                         score_v[pl.ds(slot * CH + o, 16)])

    @pl.loop(0, N_FULL // 3)
    def _(i):
        for b in range(3):
            wait(b, 3 * i + b, sems[b])
            compute(b, CH // 16, 16)

            @pl.when(i < N_FULL // 3 - 1)
            def _():
                start(b, 3 * i + b + 3, sems[b])

    off = base + N_FULL * CH
    pltpu.sync_copy(stat_hbm.at[pl.ds(off, REM)], stat_v.at[pl.ds(0, REM)])
    pltpu.sync_copy(dyn_hbm.at[pl.ds(off, REM)], dyn_v.at[pl.ds(0, REM)])
    pltpu.sync_copy(score_hbm.at[pl.ds(off, REM)], score_v.at[pl.ds(0, REM)])
    compute(0, REM // 16, 4)

    @pl.when(is_hi)
    def _():
        off2 = base + N_FULL * CH + REM
        pltpu.sync_copy(stat_hbm.at[pl.ds(off2, 16)], stat_v.at[pl.ds(0, 16)])
        pltpu.sync_copy(dyn_hbm.at[pl.ds(off2, 16)], dyn_v.at[pl.ds(0, 16)])
        pltpu.sync_copy(score_hbm.at[pl.ds(off2, 16)], score_v.at[pl.ds(0, 16)])
        scatter_vreg(stat_v[pl.ds(0, 16)], dyn_v[pl.ds(0, 16)],
                     score_v[pl.ds(0, 16)])

    pltpu.sync_copy(hist_v, shared_v.at[idx_v], add=True)
    plsc.subcore_barrier()
    pltpu.sync_copy(shared_v.at[pl.ds(sbase, 48)],
                    out_hbm.at[core, pl.ds(sbase, 48)])

    @pl.when(sid >= 14)
    def _():
        pltpu.sync_copy(shared_v.at[pl.ds(sbase + 48, 8)],
                        out_hbm.at[core, pl.ds(sbase + 48, 8)])


@jax.jit
def _sc_hist(stat, dyn, score):
    mesh = plsc.VectorSubcoreMesh(core_axis_name="c", subcore_axis_name="s")
    cp = pltpu.CompilerParams()
    if "needs_layout_passes" in pltpu.CompilerParams.__dataclass_fields__:
        cp = dataclasses.replace(cp, needs_layout_passes=False)
    f = pl.kernel(
        _sc_hist_body,
        out_type=jax.ShapeDtypeStruct((2, ROWS, 128), jnp.float32),
        mesh=mesh,
        scratch_types=[
            pltpu.VMEM((ROWS, 128), jnp.float32),
            pltpu.VMEM((3 * CH,), jnp.float32),
            pltpu.VMEM((3 * CH,), jnp.float32),
            pltpu.VMEM((3 * CH,), jnp.float32),
            pltpu.VMEM_SHARED((ROWS, 128), jnp.float32),
            pltpu.VMEM((ROWS,), jnp.int32),
            pltpu.SemaphoreType.DMA,
            pltpu.SemaphoreType.DMA,
            pltpu.SemaphoreType.DMA,
        ],
        compiler_params=cp,
    )
    return f(stat, dyn, score)


def _tc_post_body(ph_ref, mai_ref, out_ref):
    h = jnp.sum(ph_ref[...], axis=0)
    mai = mai_ref[...] * np.float32(CUW) + (np.float32(1.0 - CUW) * np.float32(W_IMP)) * h

    ii = lax.broadcasted_iota(jnp.int32, (128, 128), 0)
    jj = lax.broadcasted_iota(jnp.int32, (128, 128), 1)
    upper = (ii <= jj).astype(jnp.float32)
    w = lax.dot_general(mai, upper, (((1,), (0,)), ((), ())),
                        preferred_element_type=jnp.float32,
                        precision=lax.Precision.HIGHEST)

    rr = lax.broadcasted_iota(jnp.int32, (ROWS, ROWS), 0)
    cc = lax.broadcasted_iota(jnp.int32, (ROWS, ROWS), 1)
    lstrict = (cc < rr).astype(jnp.float32)
    s_b = jnp.broadcast_to(w[:, 127:128], (ROWS, 128))
    p = lax.dot_general(lstrict, s_b, (((1,), (0,)), ((), ())),
                        preferred_element_type=jnp.float32,
                        precision=lax.Precision.HIGHEST)
    c = w + p

    r2 = lax.broadcasted_iota(jnp.int32, (ROWS, 128), 0)
    l2 = lax.broadcasted_iota(jnp.int32, (ROWS, 128), 1)
    k = r2 * 128 + l2
    valid = k < RES
    cv = jnp.where(valid, c, jnp.float32(jnp.inf))
    best = jnp.minimum(jnp.min(cv), jnp.float32(0.0))
    eq = cv == best
    cnt = jnp.sum(eq.astype(jnp.float32)) + (best == 0.0).astype(jnp.float32)
    idxsum = jnp.sum(jnp.where(eq, (k + 1).astype(jnp.float32), jnp.float32(0.0)))
    avg = idxsum / cnt
    out_ref[...] = jnp.broadcast_to(avg * np.float32(1.0) / np.float32(RES), (1, 1))


@jax.jit
def _tc_post(part, mai_pad):
    return pl.pallas_call(
        _tc_post_body,
        out_shape=jax.ShapeDtypeStruct((1, 1), jnp.float32),
    )(part.reshape(-1, ROWS, 128), mai_pad)


def kernel(epes_stat_flow, epes_dyn_flow, moving_mask, dynamicness_scores,
           moving_average_importance, training=True):
    part = _sc_hist(epes_stat_flow, epes_dyn_flow, dynamicness_scores)
    mai_pad = jnp.pad(moving_average_importance, (0, HIST_PAD - RES)).reshape(ROWS, 128)
    out = _tc_post(part, mai_pad)
    return out[0, 0]
